# baseline (speedup 1.0000x reference)
import functools

import jax
import jax.numpy as jnp
from jax.experimental import pallas as pl
from jax.experimental.pallas import tpu as pltpu

N = 10000
E = 160000
D = 256
NPAD = 10240


def _linear_body(x_ref, w_ref, b_ref, o_ref, *, act):
    acc = jnp.dot(x_ref[...], w_ref[...], preferred_element_type=jnp.float32)
    acc = acc + b_ref[...]
    if act == "relu":
        acc = jnp.maximum(acc, 0.0)
    elif act == "lrelu":
        acc = jnp.where(acc > 0, acc, 0.01 * acc)
    o_ref[...] = acc


@functools.partial(jax.jit, static_argnames=("act", "bn"))
def _linear(x, wt, b, act="none", bn=128):
    np_, k = x.shape
    m = wt.shape[1]
    grid = (np_ // bn,)
    return pl.pallas_call(
        functools.partial(_linear_body, act=act),
        grid=grid,
        in_specs=[
            pl.BlockSpec((bn, k), lambda i: (i, 0)),
            pl.BlockSpec((k, m), lambda i: (0, 0)),
            pl.BlockSpec((1, m), lambda i: (0, 0)),
        ],
        out_specs=pl.BlockSpec((bn, m), lambda i: (i, 0)),
        out_shape=jax.ShapeDtypeStruct((np_, m), jnp.float32),
    )(x, wt, b)


def kernel(gate_types, edge_index, edge_attr, embed_table, W1_0, W2_0, b2_0,
           W1_1, W2_1, b2_1, W1_2, W2_2, b2_2, W1_3, W2_3, b2_3,
           A1, ab1, A2, ab2, C1, cb1, C2, cb2):
    W1s = [W1_0, W1_1, W1_2, W1_3]
    W2s = [W2_0, W2_1, W2_2, W2_3]
    b2s = [b2_0, b2_1, b2_2, b2_3]
    src = edge_index[0]
    dst = edge_index[1]

    deg = jax.ops.segment_sum(jnp.ones((E,), jnp.float32), dst, num_segments=N)
    inv_deg = 1.0 / jnp.clip(deg, 1.0, None)

    h = jnp.take(embed_table, gate_types, axis=0)
    h = jnp.pad(h, ((0, NPAD - N), (0, 0)))

    for li, (W1, W2, b2) in enumerate(zip(W1s, W2s, b2s)):
        ind = W1.shape[1] - 3
        W1h = W1[:, :ind]
        W1e = W1[:, ind:]
        x = _linear(h, W1h.T, jnp.zeros((1, D), jnp.float32))
        eb = edge_attr @ W1e.T
        tmp = x[src] + eb
        tmp = jnp.where(tmp > 0, tmp, 0.01 * tmp)
        s = jax.ops.segment_sum(tmp, dst, num_segments=N)
        h_n = s * inv_deg[:, None]
        h_n = jnp.pad(h_n, ((0, NPAD - N), (0, 0)))
        cat = jnp.concatenate([h, h_n], axis=1)
        h = _linear(cat, W2.T, b2[None, :], act="relu")

    a1 = _linear(h, A1.T, ab1[None, :], act="relu")
    logits = _linear(a1, A2.T, ab2[None, :])
    c1 = _linear(h, C1.T, cb1[None, :], act="relu")
    c2w = jnp.pad(C2.T, ((0, 0), (0, 127)))
    c2b = jnp.pad(cb2[None, :], ((0, 0), (0, 127)))
    vs = _linear(c1, c2w, c2b)[:N, 0]
    return (vs, logits[:N])


# speedup vs baseline: 1.0003x; 1.0003x over previous
import functools

import jax
import jax.numpy as jnp
from jax.experimental import pallas as pl

N = 10000
E = 160000
D = 256
NPAD = 10240


def _linear_body(x_ref, w_ref, b_ref, o_ref, *, act):
    acc = jnp.dot(x_ref[...], w_ref[...], preferred_element_type=jnp.float32)
    acc = acc + b_ref[...]
    if act == "relu":
        acc = jnp.maximum(acc, 0.0)
    elif act == "lrelu":
        acc = jnp.where(acc > 0, acc, 0.01 * acc)
    o_ref[...] = acc


@functools.partial(jax.jit, static_argnames=("act", "bn"))
def _linear(x, wt, b, act="none", bn=128):
    np_, k = x.shape
    m = wt.shape[1]
    grid = (np_ // bn,)
    return pl.pallas_call(
        functools.partial(_linear_body, act=act),
        grid=grid,
        in_specs=[
            pl.BlockSpec((bn, k), lambda i: (i, 0)),
            pl.BlockSpec((k, m), lambda i: (0, 0)),
            pl.BlockSpec((1, m), lambda i: (0, 0)),
        ],
        out_specs=pl.BlockSpec((bn, m), lambda i: (i, 0)),
        out_shape=jax.ShapeDtypeStruct((np_, m), jnp.float32),
    )(x, wt, b)


def kernel(gate_types, edge_index, edge_attr, embed_table, W1_0, W2_0, b2_0,
           W1_1, W2_1, b2_1, W1_2, W2_2, b2_2, W1_3, W2_3, b2_3,
           A1, ab1, A2, ab2, C1, cb1, C2, cb2):
    W1s = [W1_0, W1_1, W1_2, W1_3]
    W2s = [W2_0, W2_1, W2_2, W2_3]
    b2s = [b2_0, b2_1, b2_2, b2_3]
    src = edge_index[0]
    dst = edge_index[1]

    deg = jax.ops.segment_sum(jnp.ones((E,), jnp.float32), dst, num_segments=N)
    inv_deg = 1.0 / jnp.clip(deg, 1.0, None)

    h = jnp.take(embed_table, gate_types, axis=0)
    h = jnp.pad(h, ((0, NPAD - N), (0, 0)))

    for li, (W1, W2, b2) in enumerate(zip(W1s, W2s, b2s)):
        ind = W1.shape[1] - 3
        W1h = W1[:, :ind]
        W1e = W1[:, ind:]
        x = _linear(h, W1h.T, jnp.zeros((1, D), jnp.float32))
        eb = edge_attr @ W1e.T
        tmp = x[src] + eb
        tmp = jnp.where(tmp > 0, tmp, 0.01 * tmp)
        s = jax.ops.segment_sum(tmp, dst, num_segments=N)
        h_n = s * inv_deg[:, None]
        h_n = jnp.pad(h_n, ((0, NPAD - N), (0, 0)))
        cat = jnp.concatenate([h, h_n], axis=1)
        h = _linear(cat, W2.T, b2[None, :], act="relu")

    a1 = _linear(h, A1.T, ab1[None, :], act="relu")
    logits = _linear(a1, A2.T, ab2[None, :])
    c1 = _linear(h, C1.T, cb1[None, :], act="relu")
    c2w = jnp.pad(C2.T, ((0, 0), (0, 127)))
    c2b = jnp.pad(cb2[None, :], ((0, 0), (0, 127)))
    vs = _linear(c1, c2w, c2b)[:N, 0]
    return (vs, logits[:N])


# speedup vs baseline: 1.0019x; 1.0016x over previous
import functools

import jax
import jax.numpy as jnp
from jax.experimental import pallas as pl

N = 10000
E = 160000
D = 256
NPAD = 10240


def _linear_body(x_ref, w_ref, b_ref, o_ref, *, act):
    acc = jnp.dot(x_ref[...], w_ref[...], preferred_element_type=jnp.float32)
    acc = acc + b_ref[...]
    if act == "relu":
        acc = jnp.maximum(acc, 0.0)
    elif act == "lrelu":
        acc = jnp.where(acc > 0, acc, 0.01 * acc)
    o_ref[...] = acc


@functools.partial(jax.jit, static_argnames=("act", "bn"))
def _linear(x, wt, b, act="none", bn=128):
    np_, k = x.shape
    m = wt.shape[1]
    grid = (np_ // bn,)
    return pl.pallas_call(
        functools.partial(_linear_body, act=act),
        grid=grid,
        in_specs=[
            pl.BlockSpec((bn, k), lambda i: (i, 0)),
            pl.BlockSpec((k, m), lambda i: (0, 0)),
            pl.BlockSpec((1, m), lambda i: (0, 0)),
        ],
        out_specs=pl.BlockSpec((bn, m), lambda i: (i, 0)),
        out_shape=jax.ShapeDtypeStruct((np_, m), jnp.float32),
    )(x, wt, b)


def kernel(gate_types, edge_index, edge_attr, embed_table, W1_0, W2_0, b2_0,
           W1_1, W2_1, b2_1, W1_2, W2_2, b2_2, W1_3, W2_3, b2_3,
           A1, ab1, A2, ab2, C1, cb1, C2, cb2):
    W1s = [W1_0, W1_1, W1_2, W1_3]
    W2s = [W2_0, W2_1, W2_2, W2_3]
    b2s = [b2_0, b2_1, b2_2, b2_3]
    src = edge_index[0]
    dst = edge_index[1]

    deg = jax.ops.segment_sum(jnp.ones((E,), jnp.float32), dst, num_segments=N)
    inv_deg = 1.0 / jnp.clip(deg, 1.0, None)

    h = jnp.take(embed_table, gate_types, axis=0, mode="clip")
    h = jnp.pad(h, ((0, NPAD - N), (0, 0)))

    for li, (W1, W2, b2) in enumerate(zip(W1s, W2s, b2s)):
        ind = W1.shape[1] - 3
        W1h = W1[:, :ind]
        W1e = W1[:, ind:]
        x = _linear(h, W1h.T, jnp.zeros((1, D), jnp.float32))
        eb = edge_attr @ W1e.T
        tmp = jnp.take(x, src, axis=0, mode="clip") + eb
        tmp = jnp.where(tmp > 0, tmp, 0.01 * tmp)
        s = jax.ops.segment_sum(tmp, dst, num_segments=N)
        h_n = s * inv_deg[:, None]
        h_n = jnp.pad(h_n, ((0, NPAD - N), (0, 0)))
        cat = jnp.concatenate([h, h_n], axis=1)
        h = _linear(cat, W2.T, b2[None, :], act="relu")

    a1 = _linear(h, A1.T, ab1[None, :], act="relu")
    logits = _linear(a1, A2.T, ab2[None, :])
    c1 = _linear(h, C1.T, cb1[None, :], act="relu")
    c2w = jnp.pad(C2.T, ((0, 0), (0, 127)))
    c2b = jnp.pad(cb2[None, :], ((0, 0), (0, 127)))
    vs = _linear(c1, c2w, c2b)[:N, 0]
    return (vs, logits[:N])
